# Initial kernel scaffold; baseline (speedup 1.0000x reference)
import jax
import jax.numpy as jnp
from jax.experimental import pallas as pl


def kernel(actors, actor_idcs, actor_ctrs, nodes, node_idcs, node_ctrs, params):
    raise NotImplementedError("write your pallas kernel here")


# speedup vs baseline: 19.0497x; 19.0497x over previous
import functools

import jax
import jax.numpy as jnp
from jax import lax
from jax.experimental import pallas as pl
from jax.experimental.pallas import tpu as pltpu
from jax.experimental.pallas import tpu_sc as plsc

EPS = 1e-5
DIST_TH = 0.15

NA = 1000
KK = 6
NAK = NA * KK
NW = 32
ROWS_PT = 192
NAKP = NW * ROWS_PT
NNP = 5120
TILE_CAP = 8192
TOTAL_E = NW * TILE_CAP
CHUNK = 128
TILE_E = 512
EB = TILE_CAP // TILE_E
AROWS = ROWS_PT
LCAP = TILE_CAP // 16


def _relu(x):
    return jnp.maximum(x, 0.0)


def _dotT(x, w):
    return lax.dot_general(x, w, (((1,), (1,)), ((), ())),
                           preferred_element_type=jnp.float32)


def _gn(x, w, b):
    mu = jnp.mean(x, axis=1, keepdims=True)
    var = jnp.mean((x - mu) ** 2, axis=1, keepdims=True)
    return (x - mu) / jnp.sqrt(var + EPS) * w + b


def _d0(dv, w0, b0):
    return _relu(_dotT(dv, w0) + b0)


def _edge_build_body(dx_hbm, dy_hbm, ncx_hbm, ncy_hbm, zea_hbm, zi_hbm, zf_hbm,
                     ea_hbm, en_hbm, dvx_hbm, dvy_hbm, cnt_hbm,
                     dxv, dyv, ncxv, ncyv, eav, env, dvxv, dvyv, cntv):
    c = lax.axis_index("c")
    s = lax.axis_index("s")
    wid = s * 2 + c
    base_row = wid * ROWS_PT
    pltpu.sync_copy(dx_hbm.at[pl.ds(base_row, ROWS_PT)],
                    dxv.at[pl.ds(0, ROWS_PT)])
    pltpu.sync_copy(dy_hbm.at[pl.ds(base_row, ROWS_PT)],
                    dyv.at[pl.ds(0, ROWS_PT)])
    pltpu.sync_copy(ncx_hbm, ncxv)
    pltpu.sync_copy(ncy_hbm, ncyv)
    pltpu.sync_copy(zea_hbm, eav)
    pltpu.sync_copy(zi_hbm, env)
    pltpu.sync_copy(zf_hbm, dvxv)
    pltpu.sync_copy(zf_hbm, dvyv)

    th2 = jnp.float32(DIST_TH * DIST_TH)
    iota16 = lax.broadcasted_iota(jnp.int32, (16,), 0)
    lane_base = iota16 * LCAP

    def row_body(r, off_vec):
        row = base_row + r
        dx0 = dxv[pl.ds(r, 16)][0]
        dy0 = dyv[pl.ds(r, 16)][0]
        rowv = jnp.full((16,), row, jnp.int32)
        valid = row < NAK

        def chunk_body(j, ov):
            nx = ncxv[pl.ds(j * 16, 16)]
            ny = ncyv[pl.ds(j * 16, 16)]
            ddx = dx0 - nx
            ddy = dy0 - ny
            d2 = ddx * ddx + ddy * ddy
            msk = (d2 <= th2) & valid
            pos = lane_base + jnp.minimum(ov, LCAP - 1)
            nidx = iota16 + j * 16
            plsc.store_scatter(eav, [pos], rowv, mask=msk)
            plsc.store_scatter(env, [pos], nidx, mask=msk)
            plsc.store_scatter(dvxv, [pos], ddx, mask=msk)
            plsc.store_scatter(dvyv, [pos], ddy, mask=msk)
            return ov + jnp.where(msk, 1, 0).astype(jnp.int32)

        return lax.fori_loop(0, NNP // 16, chunk_body, off_vec)

    off_vec = lax.fori_loop(0, ROWS_PT, row_body, jnp.zeros((16,), jnp.int32))
    cntv[...] = jnp.minimum(off_vec, LCAP)
    base_e = wid * TILE_CAP
    pltpu.sync_copy(eav, ea_hbm.at[pl.ds(base_e, TILE_CAP)])
    pltpu.sync_copy(env, en_hbm.at[pl.ds(base_e, TILE_CAP)])
    pltpu.sync_copy(dvxv, dvx_hbm.at[pl.ds(base_e, TILE_CAP)])
    pltpu.sync_copy(dvyv, dvy_hbm.at[pl.ds(base_e, TILE_CAP)])
    pltpu.sync_copy(cntv, cnt_hbm.at[pl.ds(wid * 16, 16)])


def _edge_build(dxp, dyp, ncx, ncy):
    mesh = plsc.VectorSubcoreMesh(core_axis_name="c", subcore_axis_name="s")
    zea = jnp.full((TILE_CAP,), NAKP, jnp.int32)
    zi = jnp.zeros((TILE_CAP,), jnp.int32)
    zf = jnp.zeros((TILE_CAP,), jnp.float32)
    run = pl.kernel(
        _edge_build_body,
        mesh=mesh,
        compiler_params=pltpu.CompilerParams(needs_layout_passes=False),
        out_type=(jax.ShapeDtypeStruct((TOTAL_E,), jnp.int32),
                  jax.ShapeDtypeStruct((TOTAL_E,), jnp.int32),
                  jax.ShapeDtypeStruct((TOTAL_E,), jnp.float32),
                  jax.ShapeDtypeStruct((TOTAL_E,), jnp.float32),
                  jax.ShapeDtypeStruct((NW * 16,), jnp.int32)),
        scratch_types=[pltpu.VMEM((ROWS_PT + 16,), jnp.float32),
                       pltpu.VMEM((ROWS_PT + 16,), jnp.float32),
                       pltpu.VMEM((NNP,), jnp.float32),
                       pltpu.VMEM((NNP,), jnp.float32),
                       pltpu.VMEM((TILE_CAP,), jnp.int32),
                       pltpu.VMEM((TILE_CAP,), jnp.int32),
                       pltpu.VMEM((TILE_CAP,), jnp.float32),
                       pltpu.VMEM((TILE_CAP,), jnp.float32),
                       pltpu.VMEM((16,), jnp.int32)],
    )
    return run(dxp, dyp, ncx, ncy, zea, zi, zf)


def _gather_body(tab_hbm, en_hbm, cnt_hbm, gc_hbm, idxv, rowsv, cntv, sem):
    c = lax.axis_index("c")
    s = lax.axis_index("s")
    wid = s * 2 + c
    pltpu.sync_copy(cnt_hbm.at[pl.ds(wid * 16, 16)], cntv)
    base = wid * TILE_CAP
    cvec = cntv[...]
    for lane in range(16):
        c_l = cvec[lane]
        nch = jnp.where(c_l > 0, LCAP // CHUNK, 0)

        def chunk(j, carry, _lane=lane):
            off = base + _lane * LCAP + j * CHUNK
            pltpu.sync_copy(en_hbm.at[pl.ds(off, CHUNK)], idxv)
            pltpu.async_copy(tab_hbm.at[idxv], rowsv, sem).wait()
            pltpu.sync_copy(rowsv, gc_hbm.at[pl.ds(off, CHUNK)])
            return carry

        lax.fori_loop(0, nch, chunk, jnp.int32(0))


def _gather_rows(table, en, cnt):
    d = table.shape[1]
    mesh = plsc.VectorSubcoreMesh(core_axis_name="c", subcore_axis_name="s")
    run = pl.kernel(
        _gather_body,
        mesh=mesh,
        out_type=jax.ShapeDtypeStruct((TOTAL_E, d), jnp.float32),
        scratch_types=[pltpu.VMEM((CHUNK,), jnp.int32),
                       pltpu.VMEM((CHUNK, d), jnp.float32),
                       pltpu.VMEM((16,), jnp.int32),
                       pltpu.SemaphoreType.DMA],
    )
    return run(table, en, cnt)


def _pred_body(x_ref, ctr_ref, w1_ref, g1w_ref, g1b_ref, w2_ref, g2w_ref,
               g2b_ref, wp_ref, bp_ref, out_ref):
    x = x_ref[...]
    h = _relu(_gn(_dotT(x, w1_ref[0]), g1w_ref[0], g1b_ref[0]))
    h = _gn(_dotT(h, w2_ref[0]), g2w_ref[0], g2b_ref[0])
    h = _relu(h + x)
    out_ref[0] = _dotT(h, wp_ref[0]) + bp_ref[0] + ctr_ref[...]


def _aq_body(x_ref, a_ref, gw_ref, gb_ref, b_ref, out_ref):
    q = _relu(_gn(_dotT(x_ref[...], a_ref[...]), gw_ref[...], gb_ref[...]))
    out_ref[...] = _dotT(q, b_ref[...])


def _mm_body(x_ref, b_ref, out_ref):
    out_ref[...] = _dotT(x_ref[...], b_ref[...])


def _edge_mlp_body(cnt_ref, ea_ref, dvx_ref, dvy_ref, aq_ref, gc_ref,
                   w0t_ref, b0_ref, dlw_ref, dlgw_ref, dlgb_ref, wd_ref,
                   cgw_ref, cgb_ref, w2_ref, out_ref):
    t = pl.program_id(0)
    j = pl.program_id(1)

    @pl.when(j == 0)
    def _():
        out_ref[...] = jnp.zeros_like(out_ref)

    @pl.when(cnt_ref[t * 16 + j] > 0)
    def _():
        ea = ea_ref[...]
        rel = ea - t * AROWS
        iot = lax.broadcasted_iota(jnp.int32, (TILE_E, AROWS), 1)
        sel = (iot == rel).astype(jnp.float32)
        gq = lax.dot_general(sel, aq_ref[...], (((1,), (0,)), ((), ())),
                             preferred_element_type=jnp.float32,
                             precision=lax.Precision.HIGHEST)
        dv = jnp.concatenate([dvx_ref[...], dvy_ref[...]], axis=1)
        d0 = _d0(dv, w0t_ref[...], b0_ref[...])
        d1 = _relu(_gn(_dotT(d0, dlw_ref[...]), dlgw_ref[...], dlgb_ref[...]))
        y = _dotT(d1, wd_ref[...]) + gq + gc_ref[...]
        h = _relu(_gn(y, cgw_ref[...], cgb_ref[...]))
        o = _dotT(h, w2_ref[...])
        out_ref[...] += lax.dot_general(sel, o, (((0,), (0,)), ((), ())),
                                        preferred_element_type=jnp.float32,
                                        precision=lax.Precision.HIGHEST)


def _edge_mlp(cnt, ea2, dvx2, dvy2, aqp, gc, w):
    d = aqp.shape[1]

    def es(last):
        return pl.BlockSpec((TILE_E, last), lambda t, j, *_: (t * EB + j, 0))

    def rs(shape):
        return pl.BlockSpec(shape, lambda t, j, *_: (0, 0))

    grid_spec = pltpu.PrefetchScalarGridSpec(
        num_scalar_prefetch=1,
        grid=(NW, EB),
        in_specs=[es(1), es(1), es(1),
                  pl.BlockSpec((AROWS, d), lambda t, j, *_: (t, 0)),
                  es(d),
                  rs((d, 2)), rs((1, d)), rs((d, d)), rs((1, d)), rs((1, d)),
                  rs((d, d)), rs((1, d)), rs((1, d)), rs((d, d))],
        out_specs=pl.BlockSpec((AROWS, d), lambda t, j, *_: (t, 0)),
    )
    return pl.pallas_call(
        _edge_mlp_body,
        grid_spec=grid_spec,
        out_shape=jax.ShapeDtypeStruct((NAKP, d), jnp.float32),
    )(cnt, ea2, dvx2, dvy2, aqp, gc, w['w0t'], w['b0'], w['dlw'], w['dlgw'],
      w['dlgb'], w['wd'], w['cgw'], w['cgb'], w['w2'])


def _epi_body(x_ref, add_ref, agtw_ref, nw_ref, nb_ref, lw_ref, lgw_ref,
              lgb_ref, out_ref):
    x = x_ref[...]
    a = _dotT(x, agtw_ref[...]) + add_ref[...]
    a = _relu(_gn(a, nw_ref[...], nb_ref[...]))
    a = _gn(_dotT(a, lw_ref[...]), lgw_ref[...], lgb_ref[...])
    out_ref[...] = _relu(a + x)


def _feats_body(ctr_ref, dest_ref, x_ref, w0t_ref, b0_ref, dlw_ref, dlgw_ref,
                dlgb_ref, wad_ref, waa_ref, agw_ref, agb_ref, out_ref):
    dv = ctr_ref[...] - dest_ref[...]
    d0 = _d0(dv, w0t_ref[...], b0_ref[...])
    dist = _relu(_gn(_dotT(d0, dlw_ref[...]), dlgw_ref[...], dlgb_ref[...]))
    f = _dotT(dist, wad_ref[...]) + _dotT(x_ref[...], waa_ref[...])
    out_ref[...] = _relu(_gn(f, agw_ref[...], agb_ref[...]))


def _cls_body(x_ref, w1_ref, g1w_ref, g1b_ref, w2_ref, g2w_ref, g2b_ref,
              out_ref):
    x = x_ref[...]
    h = _relu(_gn(_dotT(x, w1_ref[...]), g1w_ref[...], g1b_ref[...]))
    h = _gn(_dotT(h, w2_ref[...]), g2w_ref[...], g2b_ref[...])
    out_ref[...] = _relu(h + x)


def _sort_body(cls_ref, reg_ref, clso_ref, rego_ref):
    s = cls_ref[...]
    kk = s.shape[1]
    mx = jnp.max(s, axis=1, keepdims=True)
    e = jnp.exp(s - mx)
    p = e / jnp.sum(e, axis=1, keepdims=True)
    iot = lax.broadcasted_iota(jnp.int32, p.shape, 1)
    work = p
    cls_cols = []
    reg_cols = []
    for _ in range(kk):
        v = jnp.max(work, axis=1, keepdims=True)
        idx = jnp.min(jnp.where(work == v, iot, kk), axis=1, keepdims=True)
        cls_cols.append(v)
        rj = jnp.zeros((p.shape[0], 2), jnp.float32)
        for k2 in range(kk):
            rj = rj + jnp.where(idx == k2, reg_ref[:, 2 * k2:2 * k2 + 2], 0.0)
        reg_cols.append(rj)
        work = jnp.where(iot == idx, -1e30, work)
    clso_ref[...] = jnp.concatenate(cls_cols, axis=1)
    rego_ref[...] = jnp.concatenate(reg_cols, axis=1)


def _full(shape):
    rank = len(shape)
    return pl.BlockSpec(shape, lambda *_: (0,) * rank)


def _pred_reg(actors, actor_ctrs, pr):
    na, d = actors.shape
    kk = pr['W1'].shape[0]

    def pk(shape):
        rank = len(shape)
        return pl.BlockSpec((1,) + shape[1:],
                            lambda k: (k,) + (0,) * (rank - 1))

    return pl.pallas_call(
        _pred_body,
        grid=(kk,),
        in_specs=[_full((na, d)), _full((na, 2)),
                  pk((kk, d, d)), pk((kk, 1, d)), pk((kk, 1, d)),
                  pk((kk, d, d)), pk((kk, 1, d)), pk((kk, 1, d)),
                  pk((kk, 2, d)), pk((kk, 1, 2))],
        out_specs=pk((kk, na, 2)),
        out_shape=jax.ShapeDtypeStruct((kk, na, 2), jnp.float32),
    )(actors, actor_ctrs, pr['W1'], pr['g1w'], pr['g1b'],
      pr['W2'], pr['g2w'], pr['g2b'], pr['Wp'], pr['bp'])


def _lin2(x, a, gw, gb, b):
    r, d = x.shape
    return pl.pallas_call(
        _aq_body,
        grid=(1,),
        in_specs=[_full((r, d)), _full((d, d)), _full((1, d)), _full((1, d)),
                  _full((d, d))],
        out_specs=_full((r, d)),
        out_shape=jax.ShapeDtypeStruct((r, d), jnp.float32),
    )(x, a, gw, gb, b)


def _mm(x, b):
    r, d = x.shape
    do = b.shape[0]
    return pl.pallas_call(
        _mm_body,
        grid=(1,),
        in_specs=[_full((r, d)), _full((do, d))],
        out_specs=_full((r, do)),
        out_shape=jax.ShapeDtypeStruct((r, do), jnp.float32),
    )(x, b)


def _epilogue(x6, add, w):
    r, d = x6.shape
    return pl.pallas_call(
        _epi_body,
        grid=(1,),
        in_specs=[_full((r, d)), _full((r, d)), _full((d, d)), _full((1, d)),
                  _full((1, d)), _full((d, d)), _full((1, d)), _full((1, d))],
        out_specs=_full((r, d)),
        out_shape=jax.ShapeDtypeStruct((r, d), jnp.float32),
    )(x6, add, w['agtw'], w['nw'], w['nb'], w['lw'], w['lgw'], w['lgb'])


def _feats(ctr6, dest_flat, x6, w):
    r, d = x6.shape
    return pl.pallas_call(
        _feats_body,
        grid=(1,),
        in_specs=[_full((r, 2)), _full((r, 2)), _full((r, d)), _full((d, 2)),
                  _full((1, d)), _full((d, d)), _full((1, d)), _full((1, d)),
                  _full((d, d)), _full((d, d)), _full((1, d)), _full((1, d))],
        out_specs=_full((r, d)),
        out_shape=jax.ShapeDtypeStruct((r, d), jnp.float32),
    )(ctr6, dest_flat, x6, w['w0t'], w['b0'], w['dlw'], w['dlgw'], w['dlgb'],
      w['wad'], w['waa'], w['agw'], w['agb'])


def _cls_scores(feats, w):
    r, d = feats.shape
    return pl.pallas_call(
        _cls_body,
        grid=(1,),
        in_specs=[_full((r, d)), _full((d, d)), _full((1, d)), _full((1, d)),
                  _full((d, d)), _full((1, d)), _full((1, d))],
        out_specs=_full((r, d)),
        out_shape=jax.ShapeDtypeStruct((r, d), jnp.float32),
    )(feats, w['W1'], w['g1w'], w['g1b'], w['W2'], w['g2w'], w['g2b'])


def _sort_head(cls2, reg_flat):
    na, kk = cls2.shape
    clso, rego = pl.pallas_call(
        _sort_body,
        grid=(1,),
        in_specs=[_full((na, kk)), _full((na, 2 * kk))],
        out_specs=[_full((na, kk)), _full((na, 2 * kk))],
        out_shape=[jax.ShapeDtypeStruct((na, kk), jnp.float32),
                   jax.ShapeDtypeStruct((na, 2 * kk), jnp.float32)],
    )(cls2, reg_flat)
    return clso, rego.reshape(na, kk, 2)


def _row(v):
    return v.reshape(1, -1)


def _att_weights(p, d):
    wc = p['ctx_l']['W']
    return {
        'w0t': p['dist_W0'],
        'b0': _row(p['dist_b0']),
        'dlw': p['dist_l']['W'],
        'dlgw': _row(p['dist_l']['gnw']),
        'dlgb': _row(p['dist_l']['gnb']),
        'wd': wc[:, :d],
        'wq': wc[:, d:2 * d],
        'wcf': wc[:, 2 * d:],
        'cgw': _row(p['ctx_l']['gnw']),
        'cgb': _row(p['ctx_l']['gnb']),
        'w2': p['ctx_W2'],
        'agtw': p['agt_W'],
        'nw': _row(p['norm_w']),
        'nb': _row(p['norm_b']),
        'lw': p['linear']['W'],
        'lgw': _row(p['linear']['gnw']),
        'lgb': _row(p['linear']['gnb']),
        'qw': p['query']['W'],
        'qgw': _row(p['query']['gnw']),
        'qgb': _row(p['query']['gnb']),
    }


def kernel(actors, actor_idcs, actor_ctrs, nodes, node_idcs, node_ctrs, params):
    actors = actors[actor_idcs]
    nodes = nodes[node_idcs]
    na, d = actors.shape
    kk = len(params['pred'])

    pr = {
        'W1': jnp.stack([pp['res']['W1'] for pp in params['pred']]),
        'g1w': jnp.stack([_row(pp['res']['gn1w']) for pp in params['pred']]),
        'g1b': jnp.stack([_row(pp['res']['gn1b']) for pp in params['pred']]),
        'W2': jnp.stack([pp['res']['W2'] for pp in params['pred']]),
        'g2w': jnp.stack([_row(pp['res']['gn2w']) for pp in params['pred']]),
        'g2b': jnp.stack([_row(pp['res']['gn2b']) for pp in params['pred']]),
        'Wp': jnp.stack([pp['W'] for pp in params['pred']]),
        'bp': jnp.stack([_row(pp['b']) for pp in params['pred']]),
    }
    reg6 = _pred_reg(actors, actor_ctrs, pr)
    dest_flat = reg6.reshape(kk * na, 2)

    dxp = jnp.pad(dest_flat[:, 0], (0, NAKP - kk * na))
    dyp = jnp.pad(dest_flat[:, 1], (0, NAKP - kk * na))
    ncx = jnp.pad(node_ctrs[:, 0], (0, NNP - node_ctrs.shape[0]),
                  constant_values=1e4)
    ncy = jnp.pad(node_ctrs[:, 1], (0, NNP - node_ctrs.shape[0]),
                  constant_values=1e4)
    ea, en, dvx, dvy, cnt = _edge_build(dxp, dyp, ncx, ncy)
    ea2 = ea.reshape(TOTAL_E, 1)
    dvx2 = dvx.reshape(TOTAL_E, 1)
    dvy2 = dvy.reshape(TOTAL_E, 1)

    ad = params['att_dest']
    w1 = _att_weights(ad['m2a'][0], d)
    w2 = _att_weights(ad['m2a'][1], d)

    actors6 = jnp.tile(actors, (kk, 1))
    pad6 = ((0, NAKP - kk * na), (0, 0))

    aq1 = _lin2(actors, w1['qw'], w1['qgw'], w1['qgb'], w1['wq'])
    aq1p = jnp.pad(jnp.tile(aq1, (kk, 1)), pad6)
    ac1 = _mm(nodes, w1['wcf'])
    gc1 = _gather_rows(ac1, en, cnt)
    add1 = _edge_mlp(cnt, ea2, dvx2, dvy2, aq1p, gc1, w1)
    a1 = _epilogue(actors6, add1[:kk * na], w1)

    aq2 = _lin2(a1, w2['qw'], w2['qgw'], w2['qgb'], w2['wq'])
    aq2p = jnp.pad(aq2, pad6)
    ac2 = _mm(nodes, w2['wcf'])
    gc2 = _gather_rows(ac2, en, cnt)
    add2 = _edge_mlp(cnt, ea2, dvx2, dvy2, aq2p, gc2, w2)
    a2 = _epilogue(a1, add2[:kk * na], w2)

    wa = ad['agt']['W']
    fw = {
        'w0t': ad['dist_W0'],
        'b0': _row(ad['dist_b0']),
        'dlw': ad['dist_l']['W'],
        'dlgw': _row(ad['dist_l']['gnw']),
        'dlgb': _row(ad['dist_l']['gnb']),
        'wad': wa[:, :d],
        'waa': wa[:, d:],
        'agw': _row(ad['agt']['gnw']),
        'agb': _row(ad['agt']['gnb']),
    }
    ctr6 = jnp.tile(actor_ctrs, (kk, 1))
    feats = _feats(ctr6, dest_flat, a2, fw)

    cw = {
        'W1': params['cls']['res']['W1'],
        'g1w': _row(params['cls']['res']['gn1w']),
        'g1b': _row(params['cls']['res']['gn1b']),
        'W2': params['cls']['res']['W2'],
        'g2w': _row(params['cls']['res']['gn2w']),
        'g2b': _row(params['cls']['res']['gn2b']),
        'Wp': params['cls']['W'],
    }
    hres = _cls_scores(feats, cw)
    scores = hres @ params['cls']['W'].T
    cls2 = scores.reshape(kk, na).T

    reg_nak = jnp.transpose(reg6, (1, 0, 2))
    cls_s, reg_s = _sort_head(cls2, reg_nak.reshape(na, 2 * kk))
    return cls_s, reg_s
